# initial kernel scaffold (unmeasured)
import jax
import jax.numpy as jnp
from jax import lax
from jax.experimental import pallas as pl
from jax.experimental.pallas import tpu as pltpu

N_DEV = 16
SCALE = 0.08838834764831843
BLK = 64
QC = 512


def kernel(x, Wq, K_ext, V_ext, Wo):
    B, Sq, Dm = x.shape
    _, HLDh = Wq.shape
    _, Skv, Hq_g, Dh = K_ext.shape
    HL = HLDh // Dh
    NQC = Sq // QC

    def compute_body(x_ref, wq_ref, k_hbm, v_hbm, wo_ref, out_ref,
                     q_scr, k_scr, v_scr, sems):
        my_i = lax.axis_index("i")
        x2d = x_ref[0]
        q_scr[...] = jnp.dot(x2d, wq_ref[...],
                             preferred_element_type=jnp.float32)
        out_ref[...] = jnp.zeros_like(out_ref)
        for h in range(HL):
            head = my_i * HL + h
            ck = pltpu.make_async_copy(
                k_hbm.at[0, :, pl.ds(head, 1), :], k_scr, sems.at[0])
            cv = pltpu.make_async_copy(
                v_hbm.at[0, :, pl.ds(head, 1), :], v_scr, sems.at[1])
            ck.start()
            cv.start()
            ck.wait()
            cv.wait()
            kh = k_scr[:, 0, :]
            vh = v_scr[:, 0, :]
            wo_blk = wo_ref[h * Dh:(h + 1) * Dh, :]
            for qc in range(NQC):
                kmax = (qc + 1) * QC
                qh = q_scr[qc * QC:(qc + 1) * QC, h * Dh:(h + 1) * Dh]
                s = lax.dot_general(
                    qh, kh[:kmax, :], (((1,), (1,)), ((), ())),
                    preferred_element_type=jnp.float32) * SCALE
                rows = (lax.broadcasted_iota(jnp.int32, (QC, kmax), 0)
                        + qc * QC) // BLK
                cols = lax.broadcasted_iota(jnp.int32, (QC, kmax), 1) // BLK
                s = jnp.where(cols <= rows, s, -1e9)
                m = jnp.max(s, axis=-1, keepdims=True)
                w = jnp.exp(s - m)
                w = w / jnp.sum(w, axis=-1, keepdims=True)
                ctx = jnp.dot(w, vh[:kmax, :],
                              preferred_element_type=jnp.float32)
                out_ref[0, qc * QC:(qc + 1) * QC, :] = (
                    out_ref[0, qc * QC:(qc + 1) * QC, :]
                    + jnp.dot(ctx, wo_blk,
                              preferred_element_type=jnp.float32))

    partial = pl.pallas_call(
        compute_body,
        out_shape=jax.ShapeDtypeStruct((B, Sq, Dm), jnp.float32),
        in_specs=[
            pl.BlockSpec(memory_space=pltpu.VMEM),
            pl.BlockSpec(memory_space=pltpu.VMEM),
            pl.BlockSpec(memory_space=pltpu.ANY),
            pl.BlockSpec(memory_space=pltpu.ANY),
            pl.BlockSpec(memory_space=pltpu.VMEM),
        ],
        out_specs=pl.BlockSpec(memory_space=pltpu.VMEM),
        scratch_shapes=[
            pltpu.VMEM((Sq, HLDh), jnp.float32),
            pltpu.VMEM((Skv, 1, Dh), jnp.float32),
            pltpu.VMEM((Skv, 1, Dh), jnp.float32),
            pltpu.SemaphoreType.DMA((2,)),
        ],
    )(x, Wq, K_ext, V_ext, Wo)

    CH = Sq // N_DEV

    def ar_body(p_ref, out_ref, rsbuf, sendbuf, agbuf, redbuf,
                rs_ss, rs_rs, ag_ss, ag_rs):
        my = lax.axis_index("i")
        left = (my + N_DEV - 1) % N_DEV
        right = (my + 1) % N_DEV

        barrier_sem = pltpu.get_barrier_semaphore()
        for nbr in (left, right):
            pl.semaphore_signal(
                barrier_sem, inc=1,
                device_id=(nbr,), device_id_type=pl.DeviceIdType.MESH)
        pl.semaphore_wait(barrier_sem, 2)

        sendbuf[0, :, :] = p_ref[0, pl.ds(my * CH, CH), :]
        for h in range(N_DEV - 1):
            rdma = pltpu.make_async_remote_copy(
                src_ref=sendbuf.at[h],
                dst_ref=rsbuf.at[h],
                send_sem=rs_ss.at[h],
                recv_sem=rs_rs.at[h],
                device_id=(right,),
                device_id_type=pl.DeviceIdType.MESH,
            )
            rdma.start()
            rdma.wait()
            src_chunk = (my + 2 * N_DEV - 1 - h) % N_DEV
            acc = rsbuf[h] + p_ref[0, pl.ds(src_chunk * CH, CH), :]
            if h < N_DEV - 2:
                sendbuf[h + 1, :, :] = acc
            else:
                redbuf[...] = acc

        own = (my + 1) % N_DEV
        out_ref[0, pl.ds(own * CH, CH), :] = redbuf[...]

        for h in range(N_DEV - 1):
            src = redbuf if h == 0 else agbuf.at[h - 1]
            rdma = pltpu.make_async_remote_copy(
                src_ref=src,
                dst_ref=agbuf.at[h],
                send_sem=ag_ss.at[h],
                recv_sem=ag_rs.at[h],
                device_id=(right,),
                device_id_type=pl.DeviceIdType.MESH,
            )
            rdma.start()
            rdma.wait()
            idx = (my + 2 * N_DEV - h) % N_DEV
            out_ref[0, pl.ds(idx * CH, CH), :] = agbuf[h]

    return pl.pallas_call(
        ar_body,
        out_shape=jax.ShapeDtypeStruct((B, Sq, Dm), jnp.float32),
        in_specs=[pl.BlockSpec(memory_space=pltpu.VMEM)],
        out_specs=pl.BlockSpec(memory_space=pltpu.VMEM),
        scratch_shapes=[
            pltpu.VMEM((N_DEV - 1, CH, Dm), jnp.float32),
            pltpu.VMEM((N_DEV - 1, CH, Dm), jnp.float32),
            pltpu.VMEM((N_DEV - 1, CH, Dm), jnp.float32),
            pltpu.VMEM((CH, Dm), jnp.float32),
            pltpu.SemaphoreType.DMA((N_DEV - 1,)),
            pltpu.SemaphoreType.DMA((N_DEV - 1,)),
            pltpu.SemaphoreType.DMA((N_DEV - 1,)),
            pltpu.SemaphoreType.DMA((N_DEV - 1,)),
        ],
        compiler_params=pltpu.CompilerParams(collective_id=0),
    )(partial)


# baseline (device time: 327037 ns/iter reference)
import jax
import jax.numpy as jnp
from jax import lax
from jax.experimental import pallas as pl
from jax.experimental.pallas import tpu as pltpu

N_DEV = 16
SCALE = 0.08838834764831843
BLK = 64
QC = 512


def kernel(x, Wq, K_ext, V_ext, Wo):
    B, Sq, Dm = x.shape
    _, HLDh = Wq.shape
    _, Skv, Hq_g, Dh = K_ext.shape
    HL = HLDh // Dh
    NQC = Sq // QC

    def compute_body(x_ref, wq_ref, k_hbm, v_hbm, wo_ref, out_ref,
                     q_scr, k_scr, v_scr, sems):
        my_i = lax.axis_index("i")
        x2d = x_ref[0]
        q_scr[...] = jnp.dot(x2d, wq_ref[...],
                             preferred_element_type=jnp.float32)
        out_ref[...] = jnp.zeros_like(out_ref)
        for h in range(HL):
            head = my_i * HL + h
            ck = pltpu.make_async_copy(
                k_hbm.at[0, :, pl.ds(head, 1), :], k_scr, sems.at[0])
            cv = pltpu.make_async_copy(
                v_hbm.at[0, :, pl.ds(head, 1), :], v_scr, sems.at[1])
            ck.start()
            cv.start()
            ck.wait()
            cv.wait()
            kh = k_scr[:, 0, :]
            vh = v_scr[:, 0, :]
            wo_blk = wo_ref[h * Dh:(h + 1) * Dh, :]
            for qc in range(NQC):
                kmax = (qc + 1) * QC
                qh = q_scr[qc * QC:(qc + 1) * QC, h * Dh:(h + 1) * Dh]
                s = lax.dot_general(
                    qh, kh[:kmax, :], (((1,), (1,)), ((), ())),
                    preferred_element_type=jnp.float32) * SCALE
                rows = (lax.broadcasted_iota(jnp.int32, (QC, kmax), 0)
                        + qc * QC) // BLK
                cols = lax.broadcasted_iota(jnp.int32, (QC, kmax), 1) // BLK
                s = jnp.where(cols <= rows, s, -1e9)
                m = jnp.max(s, axis=-1, keepdims=True)
                w = jnp.exp(s - m)
                w = w / jnp.sum(w, axis=-1, keepdims=True)
                ctx = jnp.dot(w, vh[:kmax, :],
                              preferred_element_type=jnp.float32)
                out_ref[0, qc * QC:(qc + 1) * QC, :] = (
                    out_ref[0, qc * QC:(qc + 1) * QC, :]
                    + jnp.dot(ctx, wo_blk,
                              preferred_element_type=jnp.float32))

    partial = pl.pallas_call(
        compute_body,
        out_shape=jax.ShapeDtypeStruct((B, Sq, Dm), jnp.float32),
        in_specs=[
            pl.BlockSpec(memory_space=pltpu.VMEM),
            pl.BlockSpec(memory_space=pltpu.VMEM),
            pl.BlockSpec(memory_space=pltpu.MemorySpace.HBM),
            pl.BlockSpec(memory_space=pltpu.MemorySpace.HBM),
            pl.BlockSpec(memory_space=pltpu.VMEM),
        ],
        out_specs=pl.BlockSpec(memory_space=pltpu.VMEM),
        scratch_shapes=[
            pltpu.VMEM((Sq, HLDh), jnp.float32),
            pltpu.VMEM((Skv, 1, Dh), jnp.float32),
            pltpu.VMEM((Skv, 1, Dh), jnp.float32),
            pltpu.SemaphoreType.DMA((2,)),
        ],
    )(x, Wq, K_ext, V_ext, Wo)

    CH = Sq // N_DEV

    def ar_body(p_ref, out_ref, rsbuf, sendbuf, agbuf, redbuf,
                rs_ss, rs_rs, ag_ss, ag_rs):
        my = lax.axis_index("i")
        left = (my + N_DEV - 1) % N_DEV
        right = (my + 1) % N_DEV

        barrier_sem = pltpu.get_barrier_semaphore()
        for nbr in (left, right):
            pl.semaphore_signal(
                barrier_sem, inc=1,
                device_id=(nbr,), device_id_type=pl.DeviceIdType.MESH)
        pl.semaphore_wait(barrier_sem, 2)

        sendbuf[0, :, :] = p_ref[0, pl.ds(my * CH, CH), :]
        for h in range(N_DEV - 1):
            rdma = pltpu.make_async_remote_copy(
                src_ref=sendbuf.at[h],
                dst_ref=rsbuf.at[h],
                send_sem=rs_ss.at[h],
                recv_sem=rs_rs.at[h],
                device_id=(right,),
                device_id_type=pl.DeviceIdType.MESH,
            )
            rdma.start()
            rdma.wait()
            src_chunk = (my + 2 * N_DEV - 1 - h) % N_DEV
            acc = rsbuf[h] + p_ref[0, pl.ds(src_chunk * CH, CH), :]
            if h < N_DEV - 2:
                sendbuf[h + 1, :, :] = acc
            else:
                redbuf[...] = acc

        own = (my + 1) % N_DEV
        out_ref[0, pl.ds(own * CH, CH), :] = redbuf[...]

        for h in range(N_DEV - 1):
            src = redbuf if h == 0 else agbuf.at[h - 1]
            rdma = pltpu.make_async_remote_copy(
                src_ref=src,
                dst_ref=agbuf.at[h],
                send_sem=ag_ss.at[h],
                recv_sem=ag_rs.at[h],
                device_id=(right,),
                device_id_type=pl.DeviceIdType.MESH,
            )
            rdma.start()
            rdma.wait()
            idx = (my + 2 * N_DEV - h) % N_DEV
            out_ref[0, pl.ds(idx * CH, CH), :] = agbuf[h]

    return pl.pallas_call(
        ar_body,
        out_shape=jax.ShapeDtypeStruct((B, Sq, Dm), jnp.float32),
        in_specs=[pl.BlockSpec(memory_space=pltpu.VMEM)],
        out_specs=pl.BlockSpec(memory_space=pltpu.VMEM),
        scratch_shapes=[
            pltpu.VMEM((N_DEV - 1, CH, Dm), jnp.float32),
            pltpu.VMEM((N_DEV - 1, CH, Dm), jnp.float32),
            pltpu.VMEM((N_DEV - 1, CH, Dm), jnp.float32),
            pltpu.VMEM((CH, Dm), jnp.float32),
            pltpu.SemaphoreType.DMA((N_DEV - 1,)),
            pltpu.SemaphoreType.DMA((N_DEV - 1,)),
            pltpu.SemaphoreType.DMA((N_DEV - 1,)),
            pltpu.SemaphoreType.DMA((N_DEV - 1,)),
        ],
        compiler_params=pltpu.CompilerParams(collective_id=0),
    )(partial)


# device time: 283182 ns/iter; 1.1549x vs baseline; 1.1549x over previous
import jax
import jax.numpy as jnp
from jax import lax
from jax.experimental import pallas as pl
from jax.experimental.pallas import tpu as pltpu

N_DEV = 16
SCALE = 0.08838834764831843
BLK = 64
QC = 512


def kernel(x, Wq, K_ext, V_ext, Wo):
    B, Sq, Dm = x.shape
    _, HLDh = Wq.shape
    _, Skv, Hq_g, Dh = K_ext.shape
    HL = HLDh // Dh
    NQC = Sq // QC

    def compute_body(x_ref, wq_ref, k_hbm, v_hbm, wo_ref, out_ref,
                     q_scr, k_scr, v_scr, sems):
        my_i = lax.axis_index("i")
        x2d = x_ref[0]
        q_scr[...] = jnp.dot(x2d, wq_ref[...],
                             preferred_element_type=jnp.float32)
        out_ref[...] = jnp.zeros_like(out_ref)
        for h in range(HL):
            head = my_i * HL + h
            ck = pltpu.make_async_copy(
                k_hbm.at[0, :, pl.ds(head, 1), :], k_scr, sems.at[0])
            cv = pltpu.make_async_copy(
                v_hbm.at[0, :, pl.ds(head, 1), :], v_scr, sems.at[1])
            ck.start()
            cv.start()
            ck.wait()
            cv.wait()
            kh = k_scr[:, 0, :]
            vh = v_scr[:, 0, :]
            wo_blk = wo_ref[h * Dh:(h + 1) * Dh, :]
            for qc in range(NQC):
                kmax = (qc + 1) * QC
                qh = q_scr[qc * QC:(qc + 1) * QC, h * Dh:(h + 1) * Dh]
                s = lax.dot_general(
                    qh, kh[:kmax, :], (((1,), (1,)), ((), ())),
                    preferred_element_type=jnp.float32) * SCALE
                rows = (lax.broadcasted_iota(jnp.int32, (QC, kmax), 0)
                        + qc * QC) // BLK
                cols = lax.broadcasted_iota(jnp.int32, (QC, kmax), 1) // BLK
                s = jnp.where(cols <= rows, s, -1e9)
                m = jnp.max(s, axis=-1, keepdims=True)
                w = jnp.exp(s - m)
                w = w / jnp.sum(w, axis=-1, keepdims=True)
                ctx = jnp.dot(w, vh[:kmax, :],
                              preferred_element_type=jnp.float32)
                out_ref[0, qc * QC:(qc + 1) * QC, :] = (
                    out_ref[0, qc * QC:(qc + 1) * QC, :]
                    + jnp.dot(ctx, wo_blk,
                              preferred_element_type=jnp.float32))

    partial = pl.pallas_call(
        compute_body,
        out_shape=jax.ShapeDtypeStruct((B, Sq, Dm), jnp.float32),
        in_specs=[
            pl.BlockSpec(memory_space=pltpu.VMEM),
            pl.BlockSpec(memory_space=pltpu.VMEM),
            pl.BlockSpec(memory_space=pltpu.MemorySpace.HBM),
            pl.BlockSpec(memory_space=pltpu.MemorySpace.HBM),
            pl.BlockSpec(memory_space=pltpu.VMEM),
        ],
        out_specs=pl.BlockSpec(memory_space=pltpu.VMEM),
        scratch_shapes=[
            pltpu.VMEM((Sq, HLDh), jnp.float32),
            pltpu.VMEM((Skv, 1, Dh), jnp.float32),
            pltpu.VMEM((Skv, 1, Dh), jnp.float32),
            pltpu.SemaphoreType.DMA((2,)),
        ],
    )(x, Wq, K_ext, V_ext, Wo)

    CH = Sq // N_DEV
    HD = Dm // 2

    def ar_body(p_ref, out_ref,
                rsbufR, sendbufR, agbufR, redbufR,
                rsbufL, sendbufL, agbufL, redbufL,
                rs_ssR, rs_rsR, ag_ssR, ag_rsR,
                rs_ssL, rs_rsL, ag_ssL, ag_rsL):
        my = lax.axis_index("i")
        left = (my + N_DEV - 1) % N_DEV
        right = (my + 1) % N_DEV

        barrier_sem = pltpu.get_barrier_semaphore()
        for nbr in (left, right):
            pl.semaphore_signal(
                barrier_sem, inc=1,
                device_id=(nbr,), device_id_type=pl.DeviceIdType.MESH)
        pl.semaphore_wait(barrier_sem, 2)

        sendbufR[0, :, :] = p_ref[0, pl.ds(my * CH, CH), 0:HD]
        sendbufL[0, :, :] = p_ref[0, pl.ds(my * CH, CH), HD:Dm]
        for h in range(N_DEV - 1):
            rdmaR = pltpu.make_async_remote_copy(
                src_ref=sendbufR.at[h], dst_ref=rsbufR.at[h],
                send_sem=rs_ssR.at[h], recv_sem=rs_rsR.at[h],
                device_id=(right,), device_id_type=pl.DeviceIdType.MESH)
            rdmaL = pltpu.make_async_remote_copy(
                src_ref=sendbufL.at[h], dst_ref=rsbufL.at[h],
                send_sem=rs_ssL.at[h], recv_sem=rs_rsL.at[h],
                device_id=(left,), device_id_type=pl.DeviceIdType.MESH)
            rdmaR.start()
            rdmaL.start()
            rdmaR.wait()
            rdmaL.wait()
            ckR = (my + 2 * N_DEV - 1 - h) % N_DEV
            ckL = (my + 1 + h) % N_DEV
            accR = rsbufR[h] + p_ref[0, pl.ds(ckR * CH, CH), 0:HD]
            accL = rsbufL[h] + p_ref[0, pl.ds(ckL * CH, CH), HD:Dm]
            if h < N_DEV - 2:
                sendbufR[h + 1, :, :] = accR
                sendbufL[h + 1, :, :] = accL
            else:
                redbufR[...] = accR
                redbufL[...] = accL

        ownR = (my + 1) % N_DEV
        ownL = (my + N_DEV - 1) % N_DEV
        out_ref[0, pl.ds(ownR * CH, CH), 0:HD] = redbufR[...]
        out_ref[0, pl.ds(ownL * CH, CH), HD:Dm] = redbufL[...]

        for h in range(N_DEV - 1):
            srcR = redbufR if h == 0 else agbufR.at[h - 1]
            srcL = redbufL if h == 0 else agbufL.at[h - 1]
            rdmaR = pltpu.make_async_remote_copy(
                src_ref=srcR, dst_ref=agbufR.at[h],
                send_sem=ag_ssR.at[h], recv_sem=ag_rsR.at[h],
                device_id=(right,), device_id_type=pl.DeviceIdType.MESH)
            rdmaL = pltpu.make_async_remote_copy(
                src_ref=srcL, dst_ref=agbufL.at[h],
                send_sem=ag_ssL.at[h], recv_sem=ag_rsL.at[h],
                device_id=(left,), device_id_type=pl.DeviceIdType.MESH)
            rdmaR.start()
            rdmaL.start()
            rdmaR.wait()
            rdmaL.wait()
            idxR = (my + 2 * N_DEV - h) % N_DEV
            idxL = (my + h) % N_DEV
            out_ref[0, pl.ds(idxR * CH, CH), 0:HD] = agbufR[h]
            out_ref[0, pl.ds(idxL * CH, CH), HD:Dm] = agbufL[h]

    nh = N_DEV - 1
    return pl.pallas_call(
        ar_body,
        out_shape=jax.ShapeDtypeStruct((B, Sq, Dm), jnp.float32),
        in_specs=[pl.BlockSpec(memory_space=pltpu.VMEM)],
        out_specs=pl.BlockSpec(memory_space=pltpu.VMEM),
        scratch_shapes=[
            pltpu.VMEM((nh, CH, HD), jnp.float32),
            pltpu.VMEM((nh, CH, HD), jnp.float32),
            pltpu.VMEM((nh, CH, HD), jnp.float32),
            pltpu.VMEM((CH, HD), jnp.float32),
            pltpu.VMEM((nh, CH, HD), jnp.float32),
            pltpu.VMEM((nh, CH, HD), jnp.float32),
            pltpu.VMEM((nh, CH, HD), jnp.float32),
            pltpu.VMEM((CH, HD), jnp.float32),
            pltpu.SemaphoreType.DMA((nh,)),
            pltpu.SemaphoreType.DMA((nh,)),
            pltpu.SemaphoreType.DMA((nh,)),
            pltpu.SemaphoreType.DMA((nh,)),
            pltpu.SemaphoreType.DMA((nh,)),
            pltpu.SemaphoreType.DMA((nh,)),
            pltpu.SemaphoreType.DMA((nh,)),
            pltpu.SemaphoreType.DMA((nh,)),
        ],
        compiler_params=pltpu.CompilerParams(collective_id=0),
    )(partial)


# device time: 240456 ns/iter; 1.3601x vs baseline; 1.1777x over previous
import jax
import jax.numpy as jnp
from jax import lax
from jax.experimental import pallas as pl
from jax.experimental.pallas import tpu as pltpu

N_DEV = 16
SCALE = 0.08838834764831843
BLK = 64
QC = 512


def kernel(x, Wq, K_ext, V_ext, Wo):
    B, Sq, Dm = x.shape
    _, HLDh = Wq.shape
    _, Skv, Hq_g, Dh = K_ext.shape
    HL = HLDh // Dh
    NQC = Sq // QC

    def compute_body(x_ref, wq_ref, k_hbm, v_hbm, wo_ref, out_ref,
                     q_scr, k_scr, v_scr, sems):
        my_i = lax.axis_index("i")
        x2d = x_ref[0]
        q_scr[...] = jnp.dot(x2d, wq_ref[...],
                             preferred_element_type=jnp.float32)
        out_ref[...] = jnp.zeros_like(out_ref)
        for h in range(HL):
            head = my_i * HL + h
            ck = pltpu.make_async_copy(
                k_hbm.at[0, :, pl.ds(head, 1), :], k_scr, sems.at[0])
            cv = pltpu.make_async_copy(
                v_hbm.at[0, :, pl.ds(head, 1), :], v_scr, sems.at[1])
            ck.start()
            cv.start()
            ck.wait()
            cv.wait()
            kh = k_scr[:, 0, :]
            vh = v_scr[:, 0, :]
            wo_blk = wo_ref[h * Dh:(h + 1) * Dh, :]
            for qc in range(NQC):
                kmax = (qc + 1) * QC
                qh = q_scr[qc * QC:(qc + 1) * QC, h * Dh:(h + 1) * Dh]
                s = lax.dot_general(
                    qh, kh[:kmax, :], (((1,), (1,)), ((), ())),
                    preferred_element_type=jnp.float32) * SCALE
                rows = (lax.broadcasted_iota(jnp.int32, (QC, kmax), 0)
                        + qc * QC) // BLK
                cols = lax.broadcasted_iota(jnp.int32, (QC, kmax), 1) // BLK
                s = jnp.where(cols <= rows, s, -1e9)
                m = jnp.max(s, axis=-1, keepdims=True)
                w = jnp.exp(s - m)
                w = w / jnp.sum(w, axis=-1, keepdims=True)
                ctx = jnp.dot(w, vh[:kmax, :],
                              preferred_element_type=jnp.float32)
                out_ref[0, qc * QC:(qc + 1) * QC, :] = (
                    out_ref[0, qc * QC:(qc + 1) * QC, :]
                    + jnp.dot(ctx, wo_blk,
                              preferred_element_type=jnp.float32))

    partial = pl.pallas_call(
        compute_body,
        out_shape=jax.ShapeDtypeStruct((B, Sq, Dm), jnp.float32),
        in_specs=[
            pl.BlockSpec(memory_space=pltpu.VMEM),
            pl.BlockSpec(memory_space=pltpu.VMEM),
            pl.BlockSpec(memory_space=pltpu.MemorySpace.HBM),
            pl.BlockSpec(memory_space=pltpu.MemorySpace.HBM),
            pl.BlockSpec(memory_space=pltpu.VMEM),
        ],
        out_specs=pl.BlockSpec(memory_space=pltpu.VMEM),
        scratch_shapes=[
            pltpu.VMEM((Sq, HLDh), jnp.float32),
            pltpu.VMEM((Skv, 1, Dh), jnp.float32),
            pltpu.VMEM((Skv, 1, Dh), jnp.float32),
            pltpu.SemaphoreType.DMA((2,)),
        ],
    )(x, Wq, K_ext, V_ext, Wo)

    CH = Sq // N_DEV
    HD = Dm // 2

    def ar_body(p_ref, out_ref,
                rsbufR, sendbufR, agbufR, redbufR,
                rsbufL, sendbufL, agbufL, redbufL,
                rs_ssR, rs_rsR, ag_ssR, ag_rsR,
                rs_ssL, rs_rsL, ag_ssL, ag_rsL):
        my = lax.axis_index("i")
        z = my // 4
        p = my % 4
        pos = jnp.where(p == 0, z,
              jnp.where(p == 3, 7 - z,
              jnp.where(p == 2, 8 + z, 15 - z)))

        def ring_logical(s):
            s = s % N_DEV
            seg = s // 4
            k = s % 4
            return jnp.where(seg == 0, 4 * k,
                   jnp.where(seg == 1, 4 * (3 - k) + 3,
                   jnp.where(seg == 2, 4 * k + 2, 4 * (3 - k) + 1)))

        left = ring_logical(pos + N_DEV - 1)
        right = ring_logical(pos + 1)

        barrier_sem = pltpu.get_barrier_semaphore()
        for nbr in (left, right):
            pl.semaphore_signal(
                barrier_sem, inc=1,
                device_id=(nbr,), device_id_type=pl.DeviceIdType.MESH)
        pl.semaphore_wait(barrier_sem, 2)

        sendbufR[0, :, :] = p_ref[0, pl.ds(pos * CH, CH), 0:HD]
        sendbufL[0, :, :] = p_ref[0, pl.ds(pos * CH, CH), HD:Dm]
        for h in range(N_DEV - 1):
            rdmaR = pltpu.make_async_remote_copy(
                src_ref=sendbufR.at[h], dst_ref=rsbufR.at[h],
                send_sem=rs_ssR.at[h], recv_sem=rs_rsR.at[h],
                device_id=(right,), device_id_type=pl.DeviceIdType.MESH)
            rdmaL = pltpu.make_async_remote_copy(
                src_ref=sendbufL.at[h], dst_ref=rsbufL.at[h],
                send_sem=rs_ssL.at[h], recv_sem=rs_rsL.at[h],
                device_id=(left,), device_id_type=pl.DeviceIdType.MESH)
            rdmaR.start()
            rdmaL.start()
            rdmaR.wait()
            rdmaL.wait()
            ckR = (pos + 2 * N_DEV - 1 - h) % N_DEV
            ckL = (pos + 1 + h) % N_DEV
            accR = rsbufR[h] + p_ref[0, pl.ds(ckR * CH, CH), 0:HD]
            accL = rsbufL[h] + p_ref[0, pl.ds(ckL * CH, CH), HD:Dm]
            if h < N_DEV - 2:
                sendbufR[h + 1, :, :] = accR
                sendbufL[h + 1, :, :] = accL
            else:
                redbufR[...] = accR
                redbufL[...] = accL

        ownR = (pos + 1) % N_DEV
        ownL = (pos + N_DEV - 1) % N_DEV
        out_ref[0, pl.ds(ownR * CH, CH), 0:HD] = redbufR[...]
        out_ref[0, pl.ds(ownL * CH, CH), HD:Dm] = redbufL[...]

        for h in range(N_DEV - 1):
            srcR = redbufR if h == 0 else agbufR.at[h - 1]
            srcL = redbufL if h == 0 else agbufL.at[h - 1]
            rdmaR = pltpu.make_async_remote_copy(
                src_ref=srcR, dst_ref=agbufR.at[h],
                send_sem=ag_ssR.at[h], recv_sem=ag_rsR.at[h],
                device_id=(right,), device_id_type=pl.DeviceIdType.MESH)
            rdmaL = pltpu.make_async_remote_copy(
                src_ref=srcL, dst_ref=agbufL.at[h],
                send_sem=ag_ssL.at[h], recv_sem=ag_rsL.at[h],
                device_id=(left,), device_id_type=pl.DeviceIdType.MESH)
            rdmaR.start()
            rdmaL.start()
            rdmaR.wait()
            rdmaL.wait()
            idxR = (pos + 2 * N_DEV - h) % N_DEV
            idxL = (pos + h) % N_DEV
            out_ref[0, pl.ds(idxR * CH, CH), 0:HD] = agbufR[h]
            out_ref[0, pl.ds(idxL * CH, CH), HD:Dm] = agbufL[h]

    nh = N_DEV - 1
    return pl.pallas_call(
        ar_body,
        out_shape=jax.ShapeDtypeStruct((B, Sq, Dm), jnp.float32),
        in_specs=[pl.BlockSpec(memory_space=pltpu.VMEM)],
        out_specs=pl.BlockSpec(memory_space=pltpu.VMEM),
        scratch_shapes=[
            pltpu.VMEM((nh, CH, HD), jnp.float32),
            pltpu.VMEM((nh, CH, HD), jnp.float32),
            pltpu.VMEM((nh, CH, HD), jnp.float32),
            pltpu.VMEM((CH, HD), jnp.float32),
            pltpu.VMEM((nh, CH, HD), jnp.float32),
            pltpu.VMEM((nh, CH, HD), jnp.float32),
            pltpu.VMEM((nh, CH, HD), jnp.float32),
            pltpu.VMEM((CH, HD), jnp.float32),
            pltpu.SemaphoreType.DMA((nh,)),
            pltpu.SemaphoreType.DMA((nh,)),
            pltpu.SemaphoreType.DMA((nh,)),
            pltpu.SemaphoreType.DMA((nh,)),
            pltpu.SemaphoreType.DMA((nh,)),
            pltpu.SemaphoreType.DMA((nh,)),
            pltpu.SemaphoreType.DMA((nh,)),
            pltpu.SemaphoreType.DMA((nh,)),
        ],
        compiler_params=pltpu.CompilerParams(collective_id=0),
    )(partial)


# device time: 197299 ns/iter; 1.6576x vs baseline; 1.2187x over previous
import jax
import jax.numpy as jnp
from jax import lax
from jax.experimental import pallas as pl
from jax.experimental.pallas import tpu as pltpu

N_DEV = 16
SCALE = 0.08838834764831843
BLK = 64
QC = 512


def kernel(x, Wq, K_ext, V_ext, Wo):
    B, Sq, Dm = x.shape
    _, HLDh = Wq.shape
    _, Skv, Hq_g, Dh = K_ext.shape
    HL = HLDh // Dh
    NQC = Sq // QC

    def compute_body(x_ref, wq_ref, k_hbm, v_hbm, wo_ref, out_ref,
                     q_scr, k_scr, v_scr, sems):
        my_i = lax.axis_index("i")
        x2d = x_ref[0]
        q_scr[...] = jnp.dot(x2d, wq_ref[...],
                             preferred_element_type=jnp.float32)
        out_ref[...] = jnp.zeros_like(out_ref)
        for h in range(HL):
            head = my_i * HL + h
            ck = pltpu.make_async_copy(
                k_hbm.at[0, :, pl.ds(head, 1), :], k_scr, sems.at[0])
            cv = pltpu.make_async_copy(
                v_hbm.at[0, :, pl.ds(head, 1), :], v_scr, sems.at[1])
            ck.start()
            cv.start()
            ck.wait()
            cv.wait()
            kh = k_scr[:, 0, :]
            vh = v_scr[:, 0, :]
            wo_blk = wo_ref[h * Dh:(h + 1) * Dh, :]
            for qc in range(NQC):
                kmax = (qc + 1) * QC
                qh = q_scr[qc * QC:(qc + 1) * QC, h * Dh:(h + 1) * Dh]
                s = lax.dot_general(
                    qh, kh[:kmax, :], (((1,), (1,)), ((), ())),
                    preferred_element_type=jnp.float32) * SCALE
                rows = (lax.broadcasted_iota(jnp.int32, (QC, kmax), 0)
                        + qc * QC) // BLK
                cols = lax.broadcasted_iota(jnp.int32, (QC, kmax), 1) // BLK
                s = jnp.where(cols <= rows, s, -1e9)
                m = jnp.max(s, axis=-1, keepdims=True)
                w = jnp.exp(s - m)
                w = w / jnp.sum(w, axis=-1, keepdims=True)
                ctx = jnp.dot(w, vh[:kmax, :],
                              preferred_element_type=jnp.float32)
                out_ref[0, qc * QC:(qc + 1) * QC, :] = (
                    out_ref[0, qc * QC:(qc + 1) * QC, :]
                    + jnp.dot(ctx, wo_blk,
                              preferred_element_type=jnp.float32))

    partial = pl.pallas_call(
        compute_body,
        out_shape=jax.ShapeDtypeStruct((B, Sq, Dm), jnp.float32),
        in_specs=[
            pl.BlockSpec(memory_space=pltpu.VMEM),
            pl.BlockSpec(memory_space=pltpu.VMEM),
            pl.BlockSpec(memory_space=pltpu.MemorySpace.HBM),
            pl.BlockSpec(memory_space=pltpu.MemorySpace.HBM),
            pl.BlockSpec(memory_space=pltpu.VMEM),
        ],
        out_specs=pl.BlockSpec(memory_space=pltpu.VMEM),
        scratch_shapes=[
            pltpu.VMEM((Sq, HLDh), jnp.float32),
            pltpu.VMEM((Skv, 1, Dh), jnp.float32),
            pltpu.VMEM((Skv, 1, Dh), jnp.float32),
            pltpu.SemaphoreType.DMA((2,)),
        ],
    )(x, Wq, K_ext, V_ext, Wo)

    CH = Sq // N_DEV
    HD = Dm // 2

    def ar_body(p_ref, out_ref,
                rsbufR, sendbufR, agbufR, redbufR,
                rsbufL, sendbufL, agbufL, redbufL,
                rs_ssR, rs_rsR, ag_ssR, ag_rsR,
                rs_ssL, rs_rsL, ag_ssL, ag_rsL):
        my = lax.axis_index("i")
        z = my // 4
        p = my % 4
        pos = jnp.where(p == 0, z,
              jnp.where(p == 3, 7 - z,
              jnp.where(p == 2, 8 + z, 15 - z)))

        def ring_logical(s):
            s = s % N_DEV
            seg = s // 4
            k = s % 4
            return jnp.where(seg == 0, 4 * k,
                   jnp.where(seg == 1, 4 * (3 - k) + 3,
                   jnp.where(seg == 2, 4 * k + 2, 4 * (3 - k) + 1)))

        left = ring_logical(pos + N_DEV - 1)
        right = ring_logical(pos + 1)

        barrier_sem = pltpu.get_barrier_semaphore()
        for nbr in (left, right):
            pl.semaphore_signal(
                barrier_sem, inc=1,
                device_id=(nbr,), device_id_type=pl.DeviceIdType.MESH)
        pl.semaphore_wait(barrier_sem, 2)

        bf16 = jnp.bfloat16
        f32 = jnp.float32
        sendbufR[0, :, :] = p_ref[0, pl.ds(pos * CH, CH), 0:HD].astype(bf16)
        sendbufL[0, :, :] = p_ref[0, pl.ds(pos * CH, CH), HD:Dm].astype(bf16)
        for h in range(N_DEV - 1):
            rdmaR = pltpu.make_async_remote_copy(
                src_ref=sendbufR.at[h], dst_ref=rsbufR.at[h],
                send_sem=rs_ssR.at[h], recv_sem=rs_rsR.at[h],
                device_id=(right,), device_id_type=pl.DeviceIdType.MESH)
            rdmaL = pltpu.make_async_remote_copy(
                src_ref=sendbufL.at[h], dst_ref=rsbufL.at[h],
                send_sem=rs_ssL.at[h], recv_sem=rs_rsL.at[h],
                device_id=(left,), device_id_type=pl.DeviceIdType.MESH)
            rdmaR.start()
            rdmaL.start()
            rdmaR.wait()
            rdmaL.wait()
            ckR = (pos + 2 * N_DEV - 1 - h) % N_DEV
            ckL = (pos + 1 + h) % N_DEV
            accR = rsbufR[h].astype(f32) + p_ref[0, pl.ds(ckR * CH, CH), 0:HD]
            accL = rsbufL[h].astype(f32) + p_ref[0, pl.ds(ckL * CH, CH), HD:Dm]
            if h < N_DEV - 2:
                sendbufR[h + 1, :, :] = accR.astype(bf16)
                sendbufL[h + 1, :, :] = accL.astype(bf16)
            else:
                redbufR[...] = accR.astype(bf16)
                redbufL[...] = accL.astype(bf16)
                ownR = (pos + 1) % N_DEV
                ownL = (pos + N_DEV - 1) % N_DEV
                out_ref[0, pl.ds(ownR * CH, CH), 0:HD] = accR
                out_ref[0, pl.ds(ownL * CH, CH), HD:Dm] = accL

        for h in range(N_DEV - 1):
            srcR = redbufR if h == 0 else agbufR.at[h - 1]
            srcL = redbufL if h == 0 else agbufL.at[h - 1]
            rdmaR = pltpu.make_async_remote_copy(
                src_ref=srcR, dst_ref=agbufR.at[h],
                send_sem=ag_ssR.at[h], recv_sem=ag_rsR.at[h],
                device_id=(right,), device_id_type=pl.DeviceIdType.MESH)
            rdmaL = pltpu.make_async_remote_copy(
                src_ref=srcL, dst_ref=agbufL.at[h],
                send_sem=ag_ssL.at[h], recv_sem=ag_rsL.at[h],
                device_id=(left,), device_id_type=pl.DeviceIdType.MESH)
            rdmaR.start()
            rdmaL.start()
            rdmaR.wait()
            rdmaL.wait()
            idxR = (pos + 2 * N_DEV - h) % N_DEV
            idxL = (pos + h) % N_DEV
            out_ref[0, pl.ds(idxR * CH, CH), 0:HD] = agbufR[h].astype(f32)
            out_ref[0, pl.ds(idxL * CH, CH), HD:Dm] = agbufL[h].astype(f32)

    nh = N_DEV - 1
    return pl.pallas_call(
        ar_body,
        out_shape=jax.ShapeDtypeStruct((B, Sq, Dm), jnp.float32),
        in_specs=[pl.BlockSpec(memory_space=pltpu.VMEM)],
        out_specs=pl.BlockSpec(memory_space=pltpu.VMEM),
        scratch_shapes=[
            pltpu.VMEM((nh, CH, HD), jnp.bfloat16),
            pltpu.VMEM((nh, CH, HD), jnp.bfloat16),
            pltpu.VMEM((nh, CH, HD), jnp.bfloat16),
            pltpu.VMEM((CH, HD), jnp.bfloat16),
            pltpu.VMEM((nh, CH, HD), jnp.bfloat16),
            pltpu.VMEM((nh, CH, HD), jnp.bfloat16),
            pltpu.VMEM((nh, CH, HD), jnp.bfloat16),
            pltpu.VMEM((CH, HD), jnp.bfloat16),
            pltpu.SemaphoreType.DMA((nh,)),
            pltpu.SemaphoreType.DMA((nh,)),
            pltpu.SemaphoreType.DMA((nh,)),
            pltpu.SemaphoreType.DMA((nh,)),
            pltpu.SemaphoreType.DMA((nh,)),
            pltpu.SemaphoreType.DMA((nh,)),
            pltpu.SemaphoreType.DMA((nh,)),
            pltpu.SemaphoreType.DMA((nh,)),
        ],
        compiler_params=pltpu.CompilerParams(collective_id=0),
    )(partial)
